# Initial kernel scaffold; baseline (speedup 1.0000x reference)
import jax
import jax.numpy as jnp
from jax.experimental import pallas as pl


def kernel(points):
    raise NotImplementedError("write your pallas kernel here")


# speedup vs baseline: 1.6566x; 1.6566x over previous
import functools

import jax
import jax.numpy as jnp
import numpy as np
from jax import lax
from jax.experimental import pallas as pl
from jax.experimental.pallas import tpu as pltpu
from jax.experimental.pallas import tpu_sc as plsc

_PC_RANGE = np.array([0.0, -40.0, -3.0, 70.4, 40.0, 1.0], dtype=np.float32)
_VOXEL_SIZE = np.array([0.05, 0.05, 0.1], dtype=np.float32)
_GRID = np.round((_PC_RANGE[3:] - _PC_RANGE[:3]) / _VOXEL_SIZE).astype(np.int32)

_N = 2_000_000
_NC, _NS = 2, 16
_NW = _NC * _NS
_P = ((_N + _NW - 1) // _NW + 7) // 8 * 8
_LAST = _N - (_NW - 1) * _P
_C = 5008
_FULL = (_P - 1) // _C

_RX, _RY, _RZ = (float(v) for v in _PC_RANGE[:3])
_SX, _SY, _SZ = (float(np.float32(1.0) / v) for v in _VOXEL_SIZE)
_GX, _GY, _GZ = (float(v) for v in _GRID)


def _chunk_compute(in_ref, out_ref):
    iota = lax.iota(jnp.int32, 16)
    c1 = jnp.full((16,), 1, jnp.int32)
    c2 = jnp.full((16,), 2, jnp.int32)

    def body(j, carry):
        r4, r3 = carry
        x = plsc.load_gather(in_ref, [r4])
        y = plsc.load_gather(in_ref, [r4 + c1])
        z = plsc.load_gather(in_ref, [r4 + c2])
        tx = (x - _RX) * _SX
        ty = (y - _RY) * _SY
        tz = (z - _RZ) * _SZ
        ok = ((tx >= 0.0) & (tx < _GX)
              & (ty >= 0.0) & (ty < _GY)
              & (tz >= 0.0) & (tz < _GZ))
        vx = jnp.where(ok, tx.astype(jnp.int32), -1)
        vy = jnp.where(ok, ty.astype(jnp.int32), -1)
        vz = jnp.where(ok, tz.astype(jnp.int32), -1)
        plsc.store_scatter(out_ref, [r3], vz)
        plsc.store_scatter(out_ref, [r3 + c1], vy)
        plsc.store_scatter(out_ref, [r3 + c2], vx)
        return r4 + 64, r3 + 48

    lax.fori_loop(0, _C // 16, body, (iota * 4, iota * 3))


@functools.partial(
    pl.kernel,
    out_type=jax.ShapeDtypeStruct((_N * 3,), jnp.int32),
    mesh=plsc.VectorSubcoreMesh(core_axis_name="c", subcore_axis_name="s"),
    scratch_types=[
        pltpu.VMEM((_C * 4,), jnp.float32),
        pltpu.VMEM((_C * 3,), jnp.int32),
    ],
    compiler_params=pltpu.CompilerParams(needs_layout_passes=False),
)
def _voxelize(points_hbm, out_hbm, in_buf, out_buf):
    wid = lax.axis_index("s") * _NC + lax.axis_index("c")
    base = wid * _P
    count = jnp.where(wid == _NW - 1, _LAST, _P)

    def chunk(k, _):
        r0 = base + jnp.where(k < _FULL, k * _C, count - _C)
        pltpu.sync_copy(points_hbm.at[pl.ds(r0 * 4, _C * 4)], in_buf)
        _chunk_compute(in_buf, out_buf)
        pltpu.sync_copy(out_buf, out_hbm.at[pl.ds(r0 * 3, _C * 3)])
        return 0

    lax.fori_loop(0, _FULL + 1, chunk, 0)


def kernel(points):
    assert points.shape == (_N, 4)
    return _voxelize(points.reshape(_N * 4)).reshape(_N, 3)


# speedup vs baseline: 27.3587x; 16.5147x over previous
import functools

import jax
import jax.numpy as jnp
import numpy as np
from jax import lax
from jax.experimental import pallas as pl
from jax.experimental.pallas import tpu as pltpu
from jax.experimental.pallas import tpu_sc as plsc

_PC_RANGE = np.array([0.0, -40.0, -3.0, 70.4, 40.0, 1.0], dtype=np.float32)
_VOXEL_SIZE = np.array([0.05, 0.05, 0.1], dtype=np.float32)
_GRID = np.round((_PC_RANGE[3:] - _PC_RANGE[:3]) / _VOXEL_SIZE).astype(np.int32)

_N = 2_000_000
_T = _N // 128
_NC, _NS = 2, 16
_NW = _NC * _NS
_PT = 496
_LAST_T = _T - (_NW - 1) * _PT
_CT = 62
_NCH = _PT // _CT
_NCH_LAST = -(-_LAST_T // _CT)

_RX, _RY, _RZ = (float(v) for v in _PC_RANGE[:3])
_SX, _SY, _SZ = (float(np.float32(1.0) / v) for v in _VOXEL_SIZE)
_GX, _GY, _GZ = (float(v) for v in _GRID)


def _chunk_compute(in_ref, out_ref):
    def tile_body(i, _):
        b = i * 512
        for g in range(8):
            o = b + g * 16
            x = in_ref[pl.ds(o, 16)]
            y = in_ref[pl.ds(o + 128, 16)]
            z = in_ref[pl.ds(o + 256, 16)]
            tx = (x - _RX) * _SX
            ty = (y - _RY) * _SY
            tz = (z - _RZ) * _SZ
            ok = ((tx >= 0.0) & (tx < _GX)
                  & (ty >= 0.0) & (ty < _GY)
                  & (tz >= 0.0) & (tz < _GZ))
            out_ref[pl.ds(o, 16)] = jnp.where(ok, tz.astype(jnp.int32), -1)
            out_ref[pl.ds(o + 128, 16)] = jnp.where(ok, ty.astype(jnp.int32), -1)
            out_ref[pl.ds(o + 256, 16)] = jnp.where(ok, tx.astype(jnp.int32), -1)
        return 0

    lax.fori_loop(0, _CT, tile_body, 0)


@functools.partial(
    pl.kernel,
    out_type=jax.ShapeDtypeStruct((_N * 4,), jnp.int32),
    mesh=plsc.VectorSubcoreMesh(core_axis_name="c", subcore_axis_name="s"),
    scratch_types=[
        pltpu.VMEM((_CT * 512,), jnp.float32),
        pltpu.VMEM((_CT * 512,), jnp.int32),
    ],
    compiler_params=pltpu.CompilerParams(needs_layout_passes=False),
)
def _voxelize(points_hbm, out_hbm, in_buf, out_buf):
    wid = lax.axis_index("s") * _NC + lax.axis_index("c")
    base = wid * _PT
    count = jnp.where(wid == _NW - 1, _LAST_T, _PT)
    n_chunks = jnp.where(wid == _NW - 1, _NCH_LAST, _NCH)

    def chunk(k, _):
        t0 = base + jnp.minimum(k * _CT, count - _CT)
        pltpu.sync_copy(points_hbm.at[pl.ds(t0 * 512, _CT * 512)], in_buf)
        _chunk_compute(in_buf, out_buf)
        pltpu.sync_copy(out_buf, out_hbm.at[pl.ds(t0 * 512, _CT * 512)])
        return 0

    lax.fori_loop(0, n_chunks, chunk, 0)


def kernel(points):
    assert points.shape == (_N, 4)
    flat = points.reshape(_T, 128, 4).transpose(0, 2, 1).reshape(_N * 4)
    out = _voxelize(flat)
    return out.reshape(_T, 4, 128).transpose(0, 2, 1).reshape(_N, 4)[:, :3]


# speedup vs baseline: 29.3476x; 1.0727x over previous
import functools

import jax
import jax.numpy as jnp
import numpy as np
from jax import lax
from jax.experimental import pallas as pl
from jax.experimental.pallas import tpu as pltpu
from jax.experimental.pallas import tpu_sc as plsc

_PC_RANGE = np.array([0.0, -40.0, -3.0, 70.4, 40.0, 1.0], dtype=np.float32)
_VOXEL_SIZE = np.array([0.05, 0.05, 0.1], dtype=np.float32)
_GRID = np.round((_PC_RANGE[3:] - _PC_RANGE[:3]) / _VOXEL_SIZE).astype(np.int32)

_N = 2_000_000
_T = _N // 128
_NC, _NS = 2, 16
_NW = _NC * _NS
_PT = 496
_LAST_T = _T - (_NW - 1) * _PT
_CT = 62
_K = _PT // _CT

_RX, _RY, _RZ = (float(v) for v in _PC_RANGE[:3])
_SX, _SY, _SZ = (float(np.float32(1.0) / v) for v in _VOXEL_SIZE)
_GX, _GY, _GZ = (float(v) for v in _GRID)


def _chunk_compute(in_ref, out_ref):
    def tile_body(i, _):
        b = i * 512
        for g in range(8):
            o = b + g * 16
            x = in_ref[pl.ds(o, 16)]
            y = in_ref[pl.ds(o + 128, 16)]
            z = in_ref[pl.ds(o + 256, 16)]
            tx = (x - _RX) * _SX
            ty = (y - _RY) * _SY
            tz = (z - _RZ) * _SZ
            ok = ((tx >= 0.0) & (tx < _GX)
                  & (ty >= 0.0) & (ty < _GY)
                  & (tz >= 0.0) & (tz < _GZ))
            out_ref[pl.ds(o, 16)] = jnp.where(ok, tz.astype(jnp.int32), -1)
            out_ref[pl.ds(o + 128, 16)] = jnp.where(ok, ty.astype(jnp.int32), -1)
            out_ref[pl.ds(o + 256, 16)] = jnp.where(ok, tx.astype(jnp.int32), -1)
        return 0

    lax.fori_loop(0, _CT, tile_body, 0)


@functools.partial(
    pl.kernel,
    out_type=jax.ShapeDtypeStruct((_N * 4,), jnp.int32),
    mesh=plsc.VectorSubcoreMesh(core_axis_name="c", subcore_axis_name="s"),
    scratch_types=[
        pltpu.VMEM((_CT * 512,), jnp.float32),
        pltpu.VMEM((_CT * 512,), jnp.float32),
        pltpu.VMEM((_CT * 512,), jnp.int32),
        pltpu.VMEM((_CT * 512,), jnp.int32),
        pltpu.SemaphoreType.DMA,
        pltpu.SemaphoreType.DMA,
        pltpu.SemaphoreType.DMA,
        pltpu.SemaphoreType.DMA,
    ],
    compiler_params=pltpu.CompilerParams(needs_layout_passes=False),
)
def _voxelize(points_hbm, out_hbm, in0, in1, out0, out1, is0, is1, os0, os1):
    wid = lax.axis_index("s") * _NC + lax.axis_index("c")
    base = wid * _PT
    count = jnp.where(wid == _NW - 1, _LAST_T, _PT)
    ins, outs, isems, osems = (in0, in1), (out0, out1), (is0, is1), (os0, os1)

    def start(k):
        e0 = (base + jnp.minimum(k * _CT, count - _CT)) * 512
        return pltpu.async_copy(
            points_hbm.at[pl.ds(e0, _CT * 512)], ins[k % 2], isems[k % 2],
        ), e0

    in_dma, e0s = {}, {}
    in_dma[0], e0s[0] = start(0)
    out_dma = {}
    for k in range(_K):
        if k + 1 < _K:
            in_dma[k + 1], e0s[k + 1] = start(k + 1)
        in_dma[k].wait()
        if k >= 2:
            out_dma[k - 2].wait()
        _chunk_compute(ins[k % 2], outs[k % 2])
        out_dma[k] = pltpu.async_copy(
            outs[k % 2], out_hbm.at[pl.ds(e0s[k], _CT * 512)], osems[k % 2],
        )
    out_dma[_K - 2].wait()
    out_dma[_K - 1].wait()


def kernel(points):
    assert points.shape == (_N, 4)
    flat = points.reshape(_T, 128, 4).transpose(0, 2, 1).reshape(_N * 4)
    out = _voxelize(flat)
    return out.reshape(_T, 4, 128).transpose(0, 2, 1).reshape(_N, 4)[:, :3]
